# baseline (speedup 1.0000x reference)
import functools

import jax
import jax.numpy as jnp
from jax.experimental import pallas as pl
from jax.experimental.pallas import tpu as pltpu


def _body(a_ref, x_ref, xn_ref, w1_ref, w2_ref, adj_ref, diff_ref,
          b1_ref, b2_ref,
          h1_ref, h2_ref, h3_ref, h4_ref, c1_ref, c2_ref,
          y_ref, *, n):
    j = pl.program_id(0)
    i = pl.program_id(1)
    steps = pl.num_programs(1)
    alpha = a_ref[0]
    h = h1_ref.shape[1]

    @pl.when(i == 0)
    def _():
        w = jnp.where(j == 0, w1_ref[...], w2_ref[...])
        y_ref[:, :h] = jnp.dot(x_ref[...], w,
                               preferred_element_type=jnp.float32
                               ).astype(jnp.bfloat16)
        y_ref[:, h:] = jnp.dot(xn_ref[...], w,
                               preferred_element_type=jnp.float32
                               ).astype(jnp.bfloat16)

    @pl.when(j == 0)
    def _():
        p = jnp.dot(adj_ref[...].astype(jnp.bfloat16), y_ref[...],
                    preferred_element_type=jnp.float32)
        zp = p[:, :h] + b1_ref[...]
        zn = p[:, h:] + b1_ref[...]
        hp = jnp.where(zp >= 0, zp, alpha * zp)
        h3_ref[...] = jnp.where(zn >= 0, zn, alpha * zn)
        h1_ref[...] = hp
        s = jnp.sum(hp, axis=0, keepdims=True)

        @pl.when(i == 0)
        def _():
            c1_ref[...] = s

        @pl.when(i > 0)
        def _():
            c1_ref[...] += s

        @pl.when(i == steps - 1)
        def _():
            c1_ref[...] = jax.nn.sigmoid(c1_ref[...] * (1.0 / n))

    @pl.when(j == 1)
    def _():
        p = jnp.dot(diff_ref[...].astype(jnp.bfloat16), y_ref[...],
                    preferred_element_type=jnp.float32)
        zp = p[:, :h] + b2_ref[...]
        zn = p[:, h:] + b2_ref[...]
        hp = jnp.where(zp >= 0, zp, alpha * zp)
        h4_ref[...] = jnp.where(zn >= 0, zn, alpha * zn)
        h2_ref[...] = hp
        s = jnp.sum(hp, axis=0, keepdims=True)

        @pl.when(i == 0)
        def _():
            c2_ref[...] = s

        @pl.when(i > 0)
        def _():
            c2_ref[...] += s

        @pl.when(i == steps - 1)
        def _():
            c2_ref[...] = jax.nn.sigmoid(c2_ref[...] * (1.0 / n))


def kernel(x, x_neg, adj, diff, W1, W2, b1, b2, a, is_sparse):
    n, f = x.shape
    h = W1.shape[1]

    bn = 200
    while n % bn != 0 or bn % 8 != 0:
        bn //= 2
    steps = n // bn
    last = steps - 1

    a2 = jnp.reshape(a, (1,)).astype(jnp.float32)
    b1r = jnp.reshape(b1, (1, h))
    b2r = jnp.reshape(b2, (1, h))

    def adj_side(j, i):
        return (jnp.where(j == 0, i, last), 0)

    def diff_side(j, i):
        return (jnp.where(j == 0, 0, i), 0)

    h1, h2, h3, h4, c1, c2 = pl.pallas_call(
        functools.partial(_body, n=float(n)),
        grid=(2, steps),
        in_specs=[
            pl.BlockSpec(memory_space=pltpu.SMEM),
            pl.BlockSpec((n, f), lambda j, i: (0, 0)),
            pl.BlockSpec((n, f), lambda j, i: (0, 0)),
            pl.BlockSpec((f, h), lambda j, i: (0, 0)),
            pl.BlockSpec((f, h), lambda j, i: (0, 0)),
            pl.BlockSpec((bn, n), adj_side),
            pl.BlockSpec((bn, n), diff_side),
            pl.BlockSpec((1, h), lambda j, i: (0, 0)),
            pl.BlockSpec((1, h), lambda j, i: (0, 0)),
        ],
        out_specs=[
            pl.BlockSpec((bn, h), adj_side),
            pl.BlockSpec((bn, h), diff_side),
            pl.BlockSpec((bn, h), adj_side),
            pl.BlockSpec((bn, h), diff_side),
            pl.BlockSpec((1, h), lambda j, i: (0, 0)),
            pl.BlockSpec((1, h), lambda j, i: (0, 0)),
        ],
        out_shape=[
            jax.ShapeDtypeStruct((n, h), jnp.float32),
            jax.ShapeDtypeStruct((n, h), jnp.float32),
            jax.ShapeDtypeStruct((n, h), jnp.float32),
            jax.ShapeDtypeStruct((n, h), jnp.float32),
            jax.ShapeDtypeStruct((1, h), jnp.float32),
            jax.ShapeDtypeStruct((1, h), jnp.float32),
        ],
        scratch_shapes=[
            pltpu.VMEM((n, 2 * h), jnp.bfloat16),
        ],
    )(a2, x, x_neg, W1, W2, adj, diff, b1r, b2r)

    return (c1[0], c2[0], h1, h2, h3, h4)


# speedup vs baseline: 1.0094x; 1.0094x over previous
import functools

import jax
import jax.numpy as jnp
from jax.experimental import pallas as pl
from jax.experimental.pallas import tpu as pltpu


def _body(a_ref, x_ref, xn_ref, w1_ref, w2_ref, adj_ref, diff_ref,
          b1_ref, b2_ref,
          h1_ref, h2_ref, h3_ref, h4_ref, c1_ref, c2_ref,
          y_ref, *, n):
    j = pl.program_id(0)
    i = pl.program_id(1)
    steps = pl.num_programs(1)
    alpha = a_ref[0]
    h = h1_ref.shape[1]

    @pl.when(i == 0)
    def _():
        w = jnp.where(j == 0, w1_ref[...], w2_ref[...])
        y_ref[:, :h] = jnp.dot(x_ref[...], w,
                               preferred_element_type=jnp.float32)
        y_ref[:, h:] = jnp.dot(xn_ref[...], w,
                               preferred_element_type=jnp.float32)

    @pl.when(j == 0)
    def _():
        p = jnp.dot(adj_ref[...], y_ref[...],
                    preferred_element_type=jnp.float32,
                    precision=jax.lax.Precision.DEFAULT)
        zp = p[:, :h] + b1_ref[...]
        zn = p[:, h:] + b1_ref[...]
        hp = jnp.where(zp >= 0, zp, alpha * zp)
        h3_ref[...] = jnp.where(zn >= 0, zn, alpha * zn)
        h1_ref[...] = hp
        s = jnp.sum(hp, axis=0, keepdims=True)

        @pl.when(i == 0)
        def _():
            c1_ref[...] = s

        @pl.when(i > 0)
        def _():
            c1_ref[...] += s

        @pl.when(i == steps - 1)
        def _():
            c1_ref[...] = jax.nn.sigmoid(c1_ref[...] * (1.0 / n))

    @pl.when(j == 1)
    def _():
        p = jnp.dot(diff_ref[...], y_ref[...],
                    preferred_element_type=jnp.float32,
                    precision=jax.lax.Precision.DEFAULT)
        zp = p[:, :h] + b2_ref[...]
        zn = p[:, h:] + b2_ref[...]
        hp = jnp.where(zp >= 0, zp, alpha * zp)
        h4_ref[...] = jnp.where(zn >= 0, zn, alpha * zn)
        h2_ref[...] = hp
        s = jnp.sum(hp, axis=0, keepdims=True)

        @pl.when(i == 0)
        def _():
            c2_ref[...] = s

        @pl.when(i > 0)
        def _():
            c2_ref[...] += s

        @pl.when(i == steps - 1)
        def _():
            c2_ref[...] = jax.nn.sigmoid(c2_ref[...] * (1.0 / n))


def kernel(x, x_neg, adj, diff, W1, W2, b1, b2, a, is_sparse):
    n, f = x.shape
    h = W1.shape[1]

    bn = 200
    while n % bn != 0 or bn % 8 != 0:
        bn //= 2
    steps = n // bn
    last = steps - 1

    a2 = jnp.reshape(a, (1,)).astype(jnp.float32)
    b1r = jnp.reshape(b1, (1, h))
    b2r = jnp.reshape(b2, (1, h))

    def adj_side(j, i):
        return (jnp.where(j == 0, i, last), 0)

    def diff_side(j, i):
        return (jnp.where(j == 0, 0, i), 0)

    h1, h2, h3, h4, c1, c2 = pl.pallas_call(
        functools.partial(_body, n=float(n)),
        grid=(2, steps),
        in_specs=[
            pl.BlockSpec(memory_space=pltpu.SMEM),
            pl.BlockSpec((n, f), lambda j, i: (0, 0)),
            pl.BlockSpec((n, f), lambda j, i: (0, 0)),
            pl.BlockSpec((f, h), lambda j, i: (0, 0)),
            pl.BlockSpec((f, h), lambda j, i: (0, 0)),
            pl.BlockSpec((bn, n), adj_side),
            pl.BlockSpec((bn, n), diff_side),
            pl.BlockSpec((1, h), lambda j, i: (0, 0)),
            pl.BlockSpec((1, h), lambda j, i: (0, 0)),
        ],
        out_specs=[
            pl.BlockSpec((bn, h), adj_side),
            pl.BlockSpec((bn, h), diff_side),
            pl.BlockSpec((bn, h), adj_side),
            pl.BlockSpec((bn, h), diff_side),
            pl.BlockSpec((1, h), lambda j, i: (0, 0)),
            pl.BlockSpec((1, h), lambda j, i: (0, 0)),
        ],
        out_shape=[
            jax.ShapeDtypeStruct((n, h), jnp.float32),
            jax.ShapeDtypeStruct((n, h), jnp.float32),
            jax.ShapeDtypeStruct((n, h), jnp.float32),
            jax.ShapeDtypeStruct((n, h), jnp.float32),
            jax.ShapeDtypeStruct((1, h), jnp.float32),
            jax.ShapeDtypeStruct((1, h), jnp.float32),
        ],
        scratch_shapes=[
            pltpu.VMEM((n, 2 * h), jnp.float32),
        ],
    )(a2, x, x_neg, W1, W2, adj, diff, b1r, b2r)

    return (c1[0], c2[0], h1, h2, h3, h4)
